# baseline (speedup 1.0000x reference)
import functools

import jax
import jax.numpy as jnp
from jax import lax
from jax.experimental import pallas as pl
from jax.experimental.pallas import tpu as pltpu
from jax.experimental.pallas import tpu_sc as plsc

D = 1024
LANES = 16
NC = 2
NS = 16
NW = NC * NS
CS = 16
RING = 5


@functools.lru_cache(maxsize=None)
def _build_sc_kernel(B, S):
    SW = S // NW
    NCH = SW // CS
    NP = NCH * B
    VPC = CS * D // LANES
    JPR = D // LANES
    mesh = plsc.VectorSubcoreMesh(core_axis_name="c", subcore_axis_name="s")

    @functools.partial(
        pl.kernel,
        mesh=mesh,
        out_type=jax.ShapeDtypeStruct((B * S, D), jnp.float32),
        scratch_types=[
            pltpu.VMEM((SW,), jnp.int32),
            pltpu.VMEM((CS, D), jnp.float32),
            pltpu.VMEM((CS, D), jnp.float32),
            pltpu.VMEM((RING * CS, D), jnp.float32),
            pltpu.SemaphoreType.DMA,
            pltpu.SemaphoreType.DMA,
            pltpu.SemaphoreType.DMA,
            pltpu.SemaphoreType.DMA,
            pltpu.SemaphoreType.DMA,
            pltpu.SemaphoreType.DMA,
            pltpu.SemaphoreType.DMA,
            pltpu.SemaphoreType.DMA,
            pltpu.SemaphoreType.DMA,
            pltpu.SemaphoreType.DMA,
            pltpu.SemaphoreType.DMA,
            pltpu.SemaphoreType.DMA,
        ],
    )
    def sc_kernel(x_hbm, ids_hbm, w_hbm, out_hbm, ids_v, embA, embB, ring,
                  se0, se1, si0, si1, si2, si3, si4,
                  so0, so1, so2, so3, so4):
        emb = (embA, embB)
        sem_e = (se0, se1)
        sem_i = (si0, si1, si2, si3, si4)
        sem_o = (so0, so1, so2, so3, so4)

        wid = lax.axis_index("s") * NC + lax.axis_index("c")
        s_base = wid * SW
        pltpu.sync_copy(ids_hbm.at[pl.ds(s_base, SW)], ids_v)

        def emb_gather(sc, start):
            e = sc % 2
            desc = pltpu.make_async_copy(
                w_hbm.at[ids_v.at[pl.ds(sc * CS, CS)]], emb[e], sem_e[e])
            if start:
                desc.start()
            return desc

        def row0(p):
            sc = p // B
            b = p % B
            return b * S + s_base + sc * CS

        def in_copy(p, r, start):
            desc = pltpu.make_async_copy(
                x_hbm.at[pl.ds(row0(p), CS)],
                ring.at[pl.ds(r * CS, CS)], sem_i[r])
            if start:
                desc.start()
            return desc

        def out_copy(p, r, start):
            desc = pltpu.make_async_copy(
                ring.at[pl.ds(r * CS, CS)],
                out_hbm.at[pl.ds(row0(p), CS)], sem_o[r])
            if start:
                desc.start()
            return desc

        emb_gather(0, start=True)
        for p in range(RING):
            in_copy(p, p, start=True)

        for p in range(NP):
            r = p % RING
            sc = p // B
            b = p % B
            if b == 0:
                emb_gather(sc, start=False).wait()
                if sc + 1 < NCH:
                    emb_gather(sc + 1, start=True)
            in_copy(p, r, start=False).wait()

            e = sc % 2
            rbase = r * CS

            def add_block(i, _):
                rr = i >> 6
                col = (i & (JPR - 1)) * LANES
                sl = pl.ds(col, LANES)
                plsc.addupdate(ring.at[rbase + rr, sl], emb[e][rr, sl])
                return 0

            lax.fori_loop(0, VPC, add_block, 0, unroll=8)

            out_copy(p, r, start=True)
            if p >= RING - 1:
                pp = p - (RING - 1)
                out_copy(pp, pp % RING, start=False).wait()
                if pp + RING < NP:
                    in_copy(pp + RING, pp % RING, start=True)

        for p in range(NP - (RING - 1), NP):
            out_copy(p, p % RING, start=False).wait()

    return sc_kernel


@jax.jit
def kernel(x, modality_ids, embed_weight):
    B, S, d = x.shape
    x2 = x.reshape(B * S, d)
    ids = modality_ids.astype(jnp.int32)
    out = _build_sc_kernel(B, S)(x2, ids, embed_weight)
    return out.reshape(B, S, d)


# speedup vs baseline: 1.1355x; 1.1355x over previous
import functools

import jax
import jax.numpy as jnp
from jax import lax
from jax.experimental import pallas as pl
from jax.experimental.pallas import tpu as pltpu
from jax.experimental.pallas import tpu_sc as plsc

D = 1024
NC = 2
NS = 16
NW = NC * NS
CS = 32
BS = 512


@functools.lru_cache(maxsize=None)
def _build_sc_gather(S):
    SW = S // NW
    NCH = SW // CS
    mesh = plsc.VectorSubcoreMesh(core_axis_name="c", subcore_axis_name="s")

    @functools.partial(
        pl.kernel,
        mesh=mesh,
        out_type=jax.ShapeDtypeStruct((S, D), jnp.float32),
        scratch_types=[
            pltpu.VMEM((SW,), jnp.int32),
            pltpu.VMEM((CS, D), jnp.float32),
            pltpu.VMEM((CS, D), jnp.float32),
            pltpu.SemaphoreType.DMA,
            pltpu.SemaphoreType.DMA,
            pltpu.SemaphoreType.DMA,
            pltpu.SemaphoreType.DMA,
        ],
    )
    def sc_gather(ids_hbm, w_hbm, emb_hbm, ids_v, bufA, bufB,
                  ge0, ge1, oe0, oe1):
        buf = (bufA, bufB)
        sem_g = (ge0, ge1)
        sem_o = (oe0, oe1)

        wid = lax.axis_index("s") * NC + lax.axis_index("c")
        s_base = wid * SW
        pltpu.sync_copy(ids_hbm.at[pl.ds(s_base, SW)], ids_v)

        def gather(c, start):
            e = c % 2
            desc = pltpu.make_async_copy(
                w_hbm.at[ids_v.at[pl.ds(c * CS, CS)]], buf[e], sem_g[e])
            if start:
                desc.start()
            return desc

        def out_copy(c, start):
            e = c % 2
            desc = pltpu.make_async_copy(
                buf[e], emb_hbm.at[pl.ds(s_base + c * CS, CS)], sem_o[e])
            if start:
                desc.start()
            return desc

        gather(0, start=True)
        for c in range(NCH):
            gather(c, start=False).wait()
            out_copy(c, start=True)
            if c >= 1:
                out_copy(c - 1, start=False).wait()
            if c + 1 < NCH:
                gather(c + 1, start=True)
        out_copy(NCH - 1, start=False).wait()

    return sc_gather


def _tc_add_body(x_ref, e_ref, o_ref):
    o_ref[...] = x_ref[...] + e_ref[...]


@functools.lru_cache(maxsize=None)
def _build_tc_add(B, S):
    return pl.pallas_call(
        _tc_add_body,
        grid=(S // BS, B),
        in_specs=[
            pl.BlockSpec((1, BS, D), lambda s, b: (b, s, 0)),
            pl.BlockSpec((1, BS, D), lambda s, b: (0, s, 0)),
        ],
        out_specs=pl.BlockSpec((1, BS, D), lambda s, b: (b, s, 0)),
        out_shape=jax.ShapeDtypeStruct((B, S, D), jnp.float32),
    )


@jax.jit
def kernel(x, modality_ids, embed_weight):
    B, S, d = x.shape
    ids = modality_ids.astype(jnp.int32)
    emb = _build_sc_gather(S)(ids, embed_weight)
    out = _build_tc_add(B, S)(x, emb.reshape(1, S, d))
    return out


# speedup vs baseline: 1.8669x; 1.6442x over previous
import functools

import jax
import jax.numpy as jnp
from jax import lax
from jax.experimental import pallas as pl
from jax.experimental.pallas import tpu as pltpu
from jax.experimental.pallas import tpu_sc as plsc

D = 1024
LANES = 16
NC = 2
NS = 16
NW = NC * NS
CS = 16
S_SC = 1024
BS = 512


@functools.lru_cache(maxsize=None)
def _build_sc_part(B, S):
    S_TC = S - S_SC
    SW = S_SC // NW
    NCH = SW // CS
    T = NCH * B
    VPC = CS * D // LANES
    JPR = D // LANES
    mesh = plsc.VectorSubcoreMesh(core_axis_name="c", subcore_axis_name="s")

    @functools.partial(
        pl.kernel,
        mesh=mesh,
        out_type=jax.ShapeDtypeStruct((B * S_SC, D), jnp.float32),
        scratch_types=[
            pltpu.VMEM((SW,), jnp.int32),
            pltpu.VMEM((CS, D), jnp.float32),
            pltpu.VMEM((CS, D), jnp.float32),
            pltpu.VMEM((CS, D), jnp.float32),
            pltpu.VMEM((CS, D), jnp.float32),
            pltpu.VMEM((CS, D), jnp.float32),
            pltpu.VMEM((CS, D), jnp.float32),
            pltpu.SemaphoreType.DMA,
            pltpu.SemaphoreType.DMA,
            pltpu.SemaphoreType.DMA,
            pltpu.SemaphoreType.DMA,
            pltpu.SemaphoreType.DMA,
            pltpu.SemaphoreType.DMA,
        ],
    )
    def sc_kernel(x_hbm, ids_hbm, w_hbm, out_hbm,
                  ids_v, emb0, emb1, in0, in1, ob0, ob1,
                  sem_e0, sem_e1, sem_i0, sem_i1, sem_o0, sem_o1):
        emb = (emb0, emb1)
        inb = (in0, in1)
        outb = (ob0, ob1)
        sem_e = (sem_e0, sem_e1)
        sem_i = (sem_i0, sem_i1)
        sem_o = (sem_o0, sem_o1)

        wid = lax.axis_index("s") * NC + lax.axis_index("c")
        w_base = wid * SW
        pltpu.sync_copy(ids_hbm.at[pl.ds(S_TC + w_base, SW)], ids_v)

        def emb_gather(c, start):
            e = c % 2
            desc = pltpu.make_async_copy(
                w_hbm.at[ids_v.at[pl.ds(c * CS, CS)]], emb[e], sem_e[e])
            if start:
                desc.start()
            return desc

        def rows(tt):
            c = tt // B
            b = tt % B
            return (b * S + S_TC + w_base + c * CS,
                    b * S_SC + w_base + c * CS)

        def in_copy(tt, k, start):
            desc = pltpu.make_async_copy(
                x_hbm.at[pl.ds(rows(tt)[0], CS)], inb[k], sem_i[k])
            if start:
                desc.start()
            return desc

        def out_copy(tt, k, start):
            desc = pltpu.make_async_copy(
                outb[k], out_hbm.at[pl.ds(rows(tt)[1], CS)], sem_o[k])
            if start:
                desc.start()
            return desc

        emb_gather(0, start=True)
        in_copy(0, 0, start=True)
        in_copy(1, 1, start=True)

        for tt in range(T):
            c = tt // B
            b = tt % B
            k = tt % 2
            e = c % 2
            if b == 0:
                emb_gather(c, start=False).wait()
                if c + 1 < NCH:
                    emb_gather(c + 1, start=True)
            in_copy(tt, k, start=False).wait()
            if tt >= 2:
                out_copy(tt - 2, k, start=False).wait()

            def add_block(i, _):
                r = i >> 6
                col = (i & (JPR - 1)) * LANES
                sl = pl.ds(col, LANES)
                outb[k][r, sl] = inb[k][r, sl] + emb[e][r, sl]
                return 0

            lax.fori_loop(0, VPC, add_block, 0, unroll=8)

            if tt + 2 < T:
                in_copy(tt + 2, k, start=True)
            out_copy(tt, k, start=True)

        out_copy(T - 2, 0, start=False).wait()
        out_copy(T - 1, 1, start=False).wait()

    return sc_kernel


def _tc_body(ids_ref, w_ref, x_ref, o_ref):
    nm = w_ref.shape[0]
    onehot = (lax.broadcasted_iota(jnp.int32, (BS, nm), 1)
              == ids_ref[...]).astype(jnp.float32)
    emb = lax.dot_general(onehot, w_ref[...], (((1,), (0,)), ((), ())),
                          preferred_element_type=jnp.float32)
    o_ref[0] = x_ref[0] + emb


@functools.lru_cache(maxsize=None)
def _build_tc_part(B, S):
    S_TC = S - S_SC
    return pl.pallas_call(
        _tc_body,
        grid=(S_TC // BS, B),
        in_specs=[
            pl.BlockSpec((BS, 1), lambda s, b: (s, 0)),
            pl.BlockSpec((5, D), lambda s, b: (0, 0)),
            pl.BlockSpec((1, BS, D), lambda s, b: (b, s, 0)),
        ],
        out_specs=pl.BlockSpec((1, BS, D), lambda s, b: (b, s, 0)),
        out_shape=jax.ShapeDtypeStruct((B, S, D), jnp.float32),
    )


@jax.jit
def kernel(x, modality_ids, embed_weight):
    B, S, d = x.shape
    S_TC = S - S_SC
    ids = modality_ids.astype(jnp.int32)
    x2 = x.reshape(B * S, d)
    sc_part = _build_sc_part(B, S)(x2, ids, embed_weight)
    tc_out = _build_tc_part(B, S)(ids.reshape(S, 1), embed_weight, x)
    return lax.dynamic_update_slice(
        tc_out, sc_part.reshape(B, S_SC, d), (0, S_TC, 0))


# speedup vs baseline: 2.1430x; 1.1479x over previous
import functools

import jax
import jax.numpy as jnp
from jax import lax
from jax.experimental import pallas as pl
from jax.experimental.pallas import tpu as pltpu
from jax.experimental.pallas import tpu_sc as plsc

D = 1024
LANES = 16
NC = 2
NS = 16
NW = NC * NS
CS = 16
S_SC = 512
BS = 512


@functools.lru_cache(maxsize=None)
def _build_sc_part(B, S):
    S_TC = S - S_SC
    SW = S_SC // NW
    NCH = SW // CS
    T = NCH * B
    VPC = CS * D // LANES
    JPR = D // LANES
    mesh = plsc.VectorSubcoreMesh(core_axis_name="c", subcore_axis_name="s")

    @functools.partial(
        pl.kernel,
        mesh=mesh,
        out_type=jax.ShapeDtypeStruct((B * S_SC, D), jnp.float32),
        scratch_types=[
            pltpu.VMEM((SW,), jnp.int32),
            pltpu.VMEM((CS, D), jnp.float32),
            pltpu.VMEM((CS, D), jnp.float32),
            pltpu.VMEM((CS, D), jnp.float32),
            pltpu.VMEM((CS, D), jnp.float32),
            pltpu.VMEM((CS, D), jnp.float32),
            pltpu.VMEM((CS, D), jnp.float32),
            pltpu.SemaphoreType.DMA,
            pltpu.SemaphoreType.DMA,
            pltpu.SemaphoreType.DMA,
            pltpu.SemaphoreType.DMA,
            pltpu.SemaphoreType.DMA,
            pltpu.SemaphoreType.DMA,
        ],
    )
    def sc_kernel(x_hbm, ids_hbm, w_hbm, out_hbm,
                  ids_v, emb0, emb1, in0, in1, ob0, ob1,
                  sem_e0, sem_e1, sem_i0, sem_i1, sem_o0, sem_o1):
        emb = (emb0, emb1)
        inb = (in0, in1)
        outb = (ob0, ob1)
        sem_e = (sem_e0, sem_e1)
        sem_i = (sem_i0, sem_i1)
        sem_o = (sem_o0, sem_o1)

        wid = lax.axis_index("s") * NC + lax.axis_index("c")
        w_base = wid * SW
        pltpu.sync_copy(ids_hbm.at[pl.ds(S_TC + w_base, SW)], ids_v)

        def emb_gather(c, start):
            e = c % 2
            desc = pltpu.make_async_copy(
                w_hbm.at[ids_v.at[pl.ds(c * CS, CS)]], emb[e], sem_e[e])
            if start:
                desc.start()
            return desc

        def rows(tt):
            c = tt // B
            b = tt % B
            return (b * S + S_TC + w_base + c * CS,
                    b * S_SC + w_base + c * CS)

        def in_copy(tt, k, start):
            desc = pltpu.make_async_copy(
                x_hbm.at[pl.ds(rows(tt)[0], CS)], inb[k], sem_i[k])
            if start:
                desc.start()
            return desc

        def out_copy(tt, k, start):
            desc = pltpu.make_async_copy(
                outb[k], out_hbm.at[pl.ds(rows(tt)[1], CS)], sem_o[k])
            if start:
                desc.start()
            return desc

        emb_gather(0, start=True)
        in_copy(0, 0, start=True)
        in_copy(1, 1, start=True)

        for tt in range(T):
            c = tt // B
            b = tt % B
            k = tt % 2
            e = c % 2
            if b == 0:
                emb_gather(c, start=False).wait()
                if c + 1 < NCH:
                    emb_gather(c + 1, start=True)
            in_copy(tt, k, start=False).wait()
            if tt >= 2:
                out_copy(tt - 2, k, start=False).wait()

            def add_block(i, _):
                r = i >> 6
                col = (i & (JPR - 1)) * LANES
                sl = pl.ds(col, LANES)
                outb[k][r, sl] = inb[k][r, sl] + emb[e][r, sl]
                return 0

            lax.fori_loop(0, VPC, add_block, 0, unroll=8)

            if tt + 2 < T:
                in_copy(tt + 2, k, start=True)
            out_copy(tt, k, start=True)

        out_copy(T - 2, 0, start=False).wait()
        out_copy(T - 1, 1, start=False).wait()

    return sc_kernel


def _tc_body(ids_ref, w_ref, x_ref, o_ref):
    nm = w_ref.shape[0]
    onehot = (lax.broadcasted_iota(jnp.int32, (BS, nm), 1)
              == ids_ref[...]).astype(jnp.float32)
    emb = lax.dot_general(onehot, w_ref[...], (((1,), (0,)), ((), ())),
                          preferred_element_type=jnp.float32)
    o_ref[0] = x_ref[0] + emb


@functools.lru_cache(maxsize=None)
def _build_tc_part(B, S):
    S_TC = S - S_SC
    return pl.pallas_call(
        _tc_body,
        grid=(S_TC // BS, B),
        in_specs=[
            pl.BlockSpec((BS, 1), lambda s, b: (s, 0)),
            pl.BlockSpec((5, D), lambda s, b: (0, 0)),
            pl.BlockSpec((1, BS, D), lambda s, b: (b, s, 0)),
        ],
        out_specs=pl.BlockSpec((1, BS, D), lambda s, b: (b, s, 0)),
        out_shape=jax.ShapeDtypeStruct((B, S, D), jnp.float32),
    )


@jax.jit
def kernel(x, modality_ids, embed_weight):
    B, S, d = x.shape
    S_TC = S - S_SC
    ids = modality_ids.astype(jnp.int32)
    x2 = x.reshape(B * S, d)
    sc_part = _build_sc_part(B, S)(x2, ids, embed_weight)
    tc_out = _build_tc_part(B, S)(ids.reshape(S, 1), embed_weight, x)
    return lax.dynamic_update_slice(
        tc_out, sc_part.reshape(B, S_SC, d), (0, S_TC, 0))


# speedup vs baseline: 2.3901x; 1.1153x over previous
import functools

import jax
import jax.numpy as jnp
from jax import lax
from jax.experimental import pallas as pl
from jax.experimental.pallas import tpu as pltpu
from jax.experimental.pallas import tpu_sc as plsc

D = 1024
LANES = 16
NC = 2
NS = 16
NW = NC * NS
CS = 8
S_SC = 256
BS = 640


@functools.lru_cache(maxsize=None)
def _build_sc_part(B, S):
    S_TC = S - S_SC
    SW = S_SC // NW
    NCH = SW // CS
    T = NCH * B
    VPC = CS * D // LANES
    JPR = D // LANES
    mesh = plsc.VectorSubcoreMesh(core_axis_name="c", subcore_axis_name="s")

    @functools.partial(
        pl.kernel,
        mesh=mesh,
        out_type=jax.ShapeDtypeStruct((B * S_SC, D), jnp.float32),
        scratch_types=[
            pltpu.VMEM((SW,), jnp.int32),
            pltpu.VMEM((CS, D), jnp.float32),
            pltpu.VMEM((CS, D), jnp.float32),
            pltpu.VMEM((CS, D), jnp.float32),
            pltpu.VMEM((CS, D), jnp.float32),
            pltpu.VMEM((CS, D), jnp.float32),
            pltpu.VMEM((CS, D), jnp.float32),
            pltpu.SemaphoreType.DMA,
            pltpu.SemaphoreType.DMA,
            pltpu.SemaphoreType.DMA,
            pltpu.SemaphoreType.DMA,
            pltpu.SemaphoreType.DMA,
            pltpu.SemaphoreType.DMA,
        ],
    )
    def sc_kernel(x_hbm, ids_hbm, w_hbm, out_hbm,
                  ids_v, emb0, emb1, in0, in1, ob0, ob1,
                  sem_e0, sem_e1, sem_i0, sem_i1, sem_o0, sem_o1):
        emb = (emb0, emb1)
        inb = (in0, in1)
        outb = (ob0, ob1)
        sem_e = (sem_e0, sem_e1)
        sem_i = (sem_i0, sem_i1)
        sem_o = (sem_o0, sem_o1)

        wid = lax.axis_index("s") * NC + lax.axis_index("c")
        w_base = wid * SW
        pltpu.sync_copy(ids_hbm.at[pl.ds(S_TC + w_base, SW)], ids_v)

        def emb_gather(c, start):
            e = c % 2
            desc = pltpu.make_async_copy(
                w_hbm.at[ids_v.at[pl.ds(c * CS, CS)]], emb[e], sem_e[e])
            if start:
                desc.start()
            return desc

        def rows(tt):
            c = tt // B
            b = tt % B
            return (b * S + S_TC + w_base + c * CS,
                    b * S_SC + w_base + c * CS)

        def in_copy(tt, k, start):
            desc = pltpu.make_async_copy(
                x_hbm.at[pl.ds(rows(tt)[0], CS)], inb[k], sem_i[k])
            if start:
                desc.start()
            return desc

        def out_copy(tt, k, start):
            desc = pltpu.make_async_copy(
                outb[k], out_hbm.at[pl.ds(rows(tt)[1], CS)], sem_o[k])
            if start:
                desc.start()
            return desc

        emb_gather(0, start=True)
        in_copy(0, 0, start=True)
        in_copy(1, 1, start=True)

        for tt in range(T):
            c = tt // B
            b = tt % B
            k = tt % 2
            e = c % 2
            if b == 0:
                emb_gather(c, start=False).wait()
                if c + 1 < NCH:
                    emb_gather(c + 1, start=True)
            in_copy(tt, k, start=False).wait()
            if tt >= 2:
                out_copy(tt - 2, k, start=False).wait()

            def add_block(i, _):
                r = i >> 6
                col = (i & (JPR - 1)) * LANES
                sl = pl.ds(col, LANES)
                outb[k][r, sl] = inb[k][r, sl] + emb[e][r, sl]
                return 0

            lax.fori_loop(0, VPC, add_block, 0, unroll=8)

            if tt + 2 < T:
                in_copy(tt + 2, k, start=True)
            out_copy(tt, k, start=True)

        out_copy(T - 2, 0, start=False).wait()
        out_copy(T - 1, 1, start=False).wait()

    return sc_kernel


def _tc_body(ids_ref, w_ref, x_ref, o_ref):
    nm = w_ref.shape[0]
    onehot = (lax.broadcasted_iota(jnp.int32, (BS, nm), 1)
              == ids_ref[...]).astype(jnp.float32)
    emb = lax.dot_general(onehot, w_ref[...], (((1,), (0,)), ((), ())),
                          preferred_element_type=jnp.float32)
    o_ref[0] = x_ref[0] + emb


@functools.lru_cache(maxsize=None)
def _build_tc_part(B, S):
    S_TC = S - S_SC
    return pl.pallas_call(
        _tc_body,
        grid=(S_TC // BS, B),
        in_specs=[
            pl.BlockSpec((BS, 1), lambda s, b: (s, 0)),
            pl.BlockSpec((5, D), lambda s, b: (0, 0)),
            pl.BlockSpec((1, BS, D), lambda s, b: (b, s, 0)),
        ],
        out_specs=pl.BlockSpec((1, BS, D), lambda s, b: (b, s, 0)),
        out_shape=jax.ShapeDtypeStruct((B, S, D), jnp.float32),
    )


@jax.jit
def kernel(x, modality_ids, embed_weight):
    B, S, d = x.shape
    S_TC = S - S_SC
    ids = modality_ids.astype(jnp.int32)
    x2 = x.reshape(B * S, d)
    sc_part = _build_sc_part(B, S)(x2, ids, embed_weight)
    tc_out = _build_tc_part(B, S)(ids.reshape(S, 1), embed_weight, x)
    return lax.dynamic_update_slice(
        tc_out, sc_part.reshape(B, S_SC, d), (0, S_TC, 0))


# speedup vs baseline: 2.4367x; 1.0195x over previous
import functools

import jax
import jax.numpy as jnp
from jax import lax
from jax.experimental import pallas as pl
from jax.experimental.pallas import tpu as pltpu
from jax.experimental.pallas import tpu_sc as plsc

D = 1024
LANES = 16
NC = 2
NS = 16
NW = NC * NS
CS = 8
S_SC = 256
BS = 768


@functools.lru_cache(maxsize=None)
def _build_sc_part(B, S):
    S_TC = S - S_SC
    SW = S_SC // NW
    NCH = SW // CS
    T = NCH * B
    VPC = CS * D // LANES
    JPR = D // LANES
    mesh = plsc.VectorSubcoreMesh(core_axis_name="c", subcore_axis_name="s")

    @functools.partial(
        pl.kernel,
        mesh=mesh,
        out_type=jax.ShapeDtypeStruct((B * S_SC, D), jnp.float32),
        scratch_types=[
            pltpu.VMEM((SW,), jnp.int32),
            pltpu.VMEM((CS, D), jnp.float32),
            pltpu.VMEM((CS, D), jnp.float32),
            pltpu.VMEM((CS, D), jnp.float32),
            pltpu.VMEM((CS, D), jnp.float32),
            pltpu.VMEM((CS, D), jnp.float32),
            pltpu.VMEM((CS, D), jnp.float32),
            pltpu.SemaphoreType.DMA,
            pltpu.SemaphoreType.DMA,
            pltpu.SemaphoreType.DMA,
            pltpu.SemaphoreType.DMA,
            pltpu.SemaphoreType.DMA,
            pltpu.SemaphoreType.DMA,
        ],
    )
    def sc_kernel(x_hbm, ids_hbm, w_hbm, out_hbm,
                  ids_v, emb0, emb1, in0, in1, ob0, ob1,
                  sem_e0, sem_e1, sem_i0, sem_i1, sem_o0, sem_o1):
        emb = (emb0, emb1)
        inb = (in0, in1)
        outb = (ob0, ob1)
        sem_e = (sem_e0, sem_e1)
        sem_i = (sem_i0, sem_i1)
        sem_o = (sem_o0, sem_o1)

        wid = lax.axis_index("s") * NC + lax.axis_index("c")
        w_base = wid * SW
        pltpu.sync_copy(ids_hbm.at[pl.ds(S_TC + w_base, SW)], ids_v)

        def emb_gather(c, start):
            e = c % 2
            desc = pltpu.make_async_copy(
                w_hbm.at[ids_v.at[pl.ds(c * CS, CS)]], emb[e], sem_e[e])
            if start:
                desc.start()
            return desc

        def rows(tt):
            c = tt // B
            b = tt % B
            return (b * S + S_TC + w_base + c * CS,
                    b * S_SC + w_base + c * CS)

        def in_copy(tt, k, start):
            desc = pltpu.make_async_copy(
                x_hbm.at[pl.ds(rows(tt)[0], CS)], inb[k], sem_i[k])
            if start:
                desc.start()
            return desc

        def out_copy(tt, k, start):
            desc = pltpu.make_async_copy(
                outb[k], out_hbm.at[pl.ds(rows(tt)[1], CS)], sem_o[k])
            if start:
                desc.start()
            return desc

        emb_gather(0, start=True)
        in_copy(0, 0, start=True)
        in_copy(1, 1, start=True)

        for tt in range(T):
            c = tt // B
            b = tt % B
            k = tt % 2
            e = c % 2
            if b == 0:
                emb_gather(c, start=False).wait()
                if c + 1 < NCH:
                    emb_gather(c + 1, start=True)
            in_copy(tt, k, start=False).wait()
            if tt >= 2:
                out_copy(tt - 2, k, start=False).wait()

            def add_block(i, _):
                r = i >> 6
                col = (i & (JPR - 1)) * LANES
                sl = pl.ds(col, LANES)
                outb[k][r, sl] = inb[k][r, sl] + emb[e][r, sl]
                return 0

            lax.fori_loop(0, VPC, add_block, 0, unroll=8)

            if tt + 2 < T:
                in_copy(tt + 2, k, start=True)
            out_copy(tt, k, start=True)

        out_copy(T - 2, 0, start=False).wait()
        out_copy(T - 1, 1, start=False).wait()

    return sc_kernel


def _tc_body(ids_ref, w_ref, x_ref, o_ref):
    nm = w_ref.shape[0]
    onehot = (lax.broadcasted_iota(jnp.int32, (BS, nm), 1)
              == ids_ref[...]).astype(jnp.float32)
    emb = lax.dot_general(onehot, w_ref[...], (((1,), (0,)), ((), ())),
                          preferred_element_type=jnp.float32)
    o_ref[0] = x_ref[0] + emb


@functools.lru_cache(maxsize=None)
def _build_tc_part(B, S):
    S_TC = S - S_SC
    return pl.pallas_call(
        _tc_body,
        grid=(S_TC // BS, B),
        in_specs=[
            pl.BlockSpec((BS, 1), lambda s, b: (s, 0)),
            pl.BlockSpec((5, D), lambda s, b: (0, 0)),
            pl.BlockSpec((1, BS, D), lambda s, b: (b, s, 0)),
        ],
        out_specs=pl.BlockSpec((1, BS, D), lambda s, b: (b, s, 0)),
        out_shape=jax.ShapeDtypeStruct((B, S, D), jnp.float32),
    )


@jax.jit
def kernel(x, modality_ids, embed_weight):
    B, S, d = x.shape
    S_TC = S - S_SC
    ids = modality_ids.astype(jnp.int32)
    x2 = x.reshape(B * S, d)
    sc_part = _build_sc_part(B, S)(x2, ids, embed_weight)
    tc_out = _build_tc_part(B, S)(ids.reshape(S, 1), embed_weight, x)
    return lax.dynamic_update_slice(
        tc_out, sc_part.reshape(B, S_SC, d), (0, S_TC, 0))
